# baseline (speedup 1.0000x reference)
import functools

import jax
import jax.numpy as jnp
from jax.experimental import pallas as pl
from jax.experimental.pallas import tpu as pltpu

def _embed_body(lens_ref, ba_ref, x_ref, w0_ref, b0_ref, w1_ref, b1_ref,
                w2_ref, b2_ref, wa_ref, e1_ref, e2_ref, sent_ref,
                m_ref, s_ref, acc_ref, *, tl, nl):
    b = pl.program_id(0)
    lt = pl.program_id(1)

    @pl.when(lt == 0)
    def _init():
        m_ref[0, 0] = -jnp.inf
        s_ref[0, 0] = jnp.float32(0.0)
        acc_ref[...] = jnp.zeros_like(acc_ref)

    x = x_ref[0].astype(jnp.bfloat16)
    e0 = jnp.dot(x, w0_ref[...], preferred_element_type=jnp.float32) + b0_ref[...]
    e1 = jnp.dot(x, w1_ref[...], preferred_element_type=jnp.float32) + b1_ref[...]
    e2 = jnp.dot(x, w2_ref[...], preferred_element_type=jnp.float32) + b2_ref[...]
    e1_ref[0] = e1
    e2_ref[0] = e2

    a = jnp.dot(e0, wa_ref[...], preferred_element_type=jnp.float32) + ba_ref[0]
    rows = jax.lax.broadcasted_iota(jnp.int32, (tl, 1), 0) + lt * tl
    a = jnp.where(rows >= lens_ref[b], jnp.float32(-1e18), a)

    m_prev = m_ref[0, 0]
    m_new = jnp.maximum(m_prev, jnp.max(a))
    alpha = jnp.exp(m_prev - m_new)
    p = jnp.exp(a - m_new)
    s_new = s_ref[0, 0] * alpha + jnp.sum(p)
    acc_new = acc_ref[...] * alpha + jnp.sum(p * e0, axis=0, keepdims=True)
    m_ref[0, 0] = m_new
    s_ref[0, 0] = s_new
    acc_ref[...] = acc_new

    @pl.when(lt == nl - 1)
    def _fin():
        sent_ref[0] = acc_new / s_new


_T_RANGE = 30.0
_WIDTH = 6.0e-5
_N_CAP = 20


def _sigmoid(x):
    return 0.5 + 0.5 * jnp.tanh(0.5 * x)


def _bisect(data, k_f, lo0, hi0, n):
    def body(_, carry):
        lo, hi = carry
        mid = 0.5 * (lo + hi)
        cnt = jnp.sum(jnp.where(data > mid, 1.0, 0.0).astype(jnp.float32),
                      axis=0, keepdims=True)
        ge = cnt >= k_f
        return jnp.where(ge, mid, lo), jnp.where(ge, hi, mid)
    return jax.lax.fori_loop(0, n, body, (lo0, hi0))


def _topk_mean(pre, len_b, l):
    c = pre.shape[1]
    rows = jax.lax.broadcasted_iota(jnp.int32, (l, 1), 0)
    pre_m = jnp.where(rows < len_b, pre, -jnp.inf)
    k_i = (len_b + jnp.int32(7)) // jnp.int32(8)
    k_f = k_i.astype(jnp.float32)
    k8_f = ((k_i + jnp.int32(7)) // jnp.int32(8)).astype(jnp.float32)

    mx = jnp.max(pre_m.reshape(l // 8, 8, c), axis=1)

    lo0 = jnp.full((1, c), -_T_RANGE, jnp.float32)
    hi0 = jnp.full((1, c), _T_RANGE, jnp.float32)
    _, t_hi = _bisect(mx, k8_f, lo0, hi0, _N_CAP)
    _, t_lo = _bisect(mx, k_f, lo0, hi0, _N_CAP)

    lo1 = t_lo - jnp.float32(2.0 * _WIDTH)
    hi1 = jnp.maximum(t_hi, lo1)
    w = jnp.max(hi1 - lo1) + jnp.float32(1e-30)
    expo = (jax.lax.bitcast_convert_type(w, jnp.int32) >> 23) - jnp.int32(126)
    n2 = jnp.clip(expo + jnp.int32(15), 0, _N_CAP)
    _, t = _bisect(pre_m, k_f, lo1, hi1, n2)

    sig = _sigmoid(pre)
    gt = pre_m > t
    cnt_gt = jnp.sum(gt.astype(jnp.float32), axis=0, keepdims=True)
    sum_gt = jnp.sum(jnp.where(gt, sig, jnp.float32(0.0)), axis=0, keepdims=True)
    t_sig = _sigmoid(t)
    return (sum_gt + (k_f - cnt_gt) * t_sig) / k_f


def _verb_body(lens_ref, e1_ref, w0_ref, w1_ref, w2_ref, bi_ref,
               out_ref, *, l):
    b = pl.program_id(0)
    e1 = e1_ref[0].astype(jnp.bfloat16)
    ym = jnp.dot(e1, w0_ref[...], preferred_element_type=jnp.float32)
    yc = jnp.dot(e1, w1_ref[...], preferred_element_type=jnp.float32)
    yp = jnp.dot(e1, w2_ref[...], preferred_element_type=jnp.float32)
    z = jnp.zeros((1, ym.shape[1]), jnp.float32)
    pre = yc + bi_ref[...] + jnp.concatenate([z, ym[:-1]], axis=0) \
             + jnp.concatenate([yp[1:], z], axis=0)
    out_ref[0] = _topk_mean(pre, lens_ref[b], l)


def _noun_body(lens_ref, e2_ref, w_ref, bi_ref, out_ref, *, l):
    b = pl.program_id(0)
    pre = jnp.dot(e2_ref[0].astype(jnp.bfloat16), w_ref[...],
                  preferred_element_type=jnp.float32) + bi_ref[...]
    out_ref[0] = _topk_mean(pre, lens_ref[b], l)


def kernel(inputs, input_lens, W0, b0, W1, b1, W2, b2, Wa, ba, cvw, cvb,
           bnvg, bnvb, bnvm, bnvv, cnw, cnb, bnng, bnnb, bnnm, bnnv):
    B, L, D_IN = inputs.shape
    D = W0.shape[1]
    VC = cvw.shape[0]
    NC = cnw.shape[0]
    TL = 512 if L % 512 == 0 else L
    NL = L // TL

    lens = input_lens.astype(jnp.int32)

    grid1 = (B, NL)
    full = lambda shp: pl.BlockSpec(shp, lambda b, t: (0,) * len(shp))
    e1, e2, sent = pl.pallas_call(
        functools.partial(_embed_body, tl=TL, nl=NL),
        grid=grid1,
        in_specs=[
            pl.BlockSpec(memory_space=pltpu.SMEM),
            pl.BlockSpec(memory_space=pltpu.SMEM),
            pl.BlockSpec((1, TL, D_IN), lambda b, t: (b, t, 0)),
            full((D_IN, D)), full((1, D)),
            full((D_IN, D)), full((1, D)),
            full((D_IN, D)), full((1, D)),
            full((D, 1)),
        ],
        out_specs=[
            pl.BlockSpec((1, TL, D), lambda b, t: (b, t, 0)),
            pl.BlockSpec((1, TL, D), lambda b, t: (b, t, 0)),
            pl.BlockSpec((1, 1, D), lambda b, t: (b, 0, 0)),
        ],
        out_shape=[
            jax.ShapeDtypeStruct((B, L, D), jnp.float32),
            jax.ShapeDtypeStruct((B, L, D), jnp.float32),
            jax.ShapeDtypeStruct((B, 1, D), jnp.float32),
        ],
        scratch_shapes=[
            pltpu.SMEM((1, 1), jnp.float32),
            pltpu.SMEM((1, 1), jnp.float32),
            pltpu.VMEM((1, D), jnp.float32),
        ],
        compiler_params=pltpu.CompilerParams(
            dimension_semantics=("parallel", "arbitrary")),
    )(lens, ba, inputs, W0.astype(jnp.bfloat16), b0.reshape(1, D),
      W1.astype(jnp.bfloat16), b1.reshape(1, D),
      W2.astype(jnp.bfloat16), b2.reshape(1, D), Wa)

    va = bnvg / jnp.sqrt(bnvv + 1e-5)
    v_scale = va.reshape(1, VC)
    v_bias = (bnvb + (cvb - bnvm) * va).reshape(1, VC)
    na = bnng / jnp.sqrt(bnnv + 1e-5)
    n_scale = na.reshape(1, NC)
    n_bias = (bnnb + (cnb - bnnm) * na).reshape(1, NC)

    wvs = (cvw * va[:, None, None])
    wv0 = wvs[:, :, 0].T.astype(jnp.bfloat16)
    wv1 = wvs[:, :, 1].T.astype(jnp.bfloat16)
    wv2 = wvs[:, :, 2].T.astype(jnp.bfloat16)
    wn = (cnw[:, :, 0] * na[:, None]).T.astype(jnp.bfloat16)

    fullb = lambda shp: pl.BlockSpec(shp, lambda b: (0,) * len(shp))
    ilv = pl.pallas_call(
        functools.partial(_verb_body, l=L),
        grid=(B,),
        in_specs=[
            pl.BlockSpec(memory_space=pltpu.SMEM),
            pl.BlockSpec((1, L, D), lambda b: (b, 0, 0)),
            fullb((D, VC)), fullb((D, VC)), fullb((D, VC)),
            fullb((1, VC)),
        ],
        out_specs=pl.BlockSpec((1, 1, VC), lambda b: (b, 0, 0)),
        out_shape=jax.ShapeDtypeStruct((B, 1, VC), jnp.float32),
        compiler_params=pltpu.CompilerParams(
            dimension_semantics=("parallel",)),
    )(lens, e1, wv0, wv1, wv2, v_bias)

    iln = pl.pallas_call(
        functools.partial(_noun_body, l=L),
        grid=(B,),
        in_specs=[
            pl.BlockSpec(memory_space=pltpu.SMEM),
            pl.BlockSpec((1, L, D), lambda b: (b, 0, 0)),
            fullb((D, NC)), fullb((1, NC)),
        ],
        out_specs=pl.BlockSpec((1, 1, NC), lambda b: (b, 0, 0)),
        out_shape=jax.ShapeDtypeStruct((B, 1, NC), jnp.float32),
        compiler_params=pltpu.CompilerParams(
            dimension_semantics=("parallel",)),
    )(lens, e2, wn, n_bias)

    return (sent.reshape(B, D), e1, e2, ilv.reshape(B, VC), iln.reshape(B, NC))


# speedup vs baseline: 2.5194x; 2.5194x over previous
import functools

import jax
import jax.numpy as jnp
from jax.experimental import pallas as pl
from jax.experimental.pallas import tpu as pltpu

def _embed_body(lens_ref, ba_ref, x_ref, w0_ref, b0_ref, w1_ref, b1_ref,
                w2_ref, b2_ref, wa_ref, e1_ref, e2_ref, sent_ref,
                m_ref, s_ref, acc_ref, *, tl, nl):
    b = pl.program_id(0)
    lt = pl.program_id(1)

    @pl.when(lt == 0)
    def _init():
        m_ref[0, 0] = -jnp.inf
        s_ref[0, 0] = jnp.float32(0.0)
        acc_ref[...] = jnp.zeros_like(acc_ref)

    x = x_ref[0].astype(jnp.bfloat16)
    e0 = jnp.dot(x, w0_ref[...], preferred_element_type=jnp.float32) + b0_ref[...]
    e1 = jnp.dot(x, w1_ref[...], preferred_element_type=jnp.float32) + b1_ref[...]
    e2 = jnp.dot(x, w2_ref[...], preferred_element_type=jnp.float32) + b2_ref[...]
    e1_ref[0] = e1
    e2_ref[0] = e2

    a = jnp.dot(e0, wa_ref[...], preferred_element_type=jnp.float32) + ba_ref[0]
    rows = jax.lax.broadcasted_iota(jnp.int32, (tl, 1), 0) + lt * tl
    a = jnp.where(rows >= lens_ref[b], jnp.float32(-1e18), a)

    m_prev = m_ref[0, 0]
    m_new = jnp.maximum(m_prev, jnp.max(a))
    alpha = jnp.exp(m_prev - m_new)
    p = jnp.exp(a - m_new)
    s_new = s_ref[0, 0] * alpha + jnp.sum(p)
    acc_new = acc_ref[...] * alpha + jnp.sum(p * e0, axis=0, keepdims=True)
    m_ref[0, 0] = m_new
    s_ref[0, 0] = s_new
    acc_ref[...] = acc_new

    @pl.when(lt == nl - 1)
    def _fin():
        sent_ref[0] = acc_new / s_new


_T_RANGE = 30.0
_N_BISECT = 15


def _sigmoid(x):
    return 0.5 + 0.5 * jnp.tanh(0.5 * x)


def _bisect(data, k_f, lo0, hi0, n):
    def body(_, carry):
        lo, hi = carry
        mid = 0.5 * (lo + hi)
        cnt = jnp.sum(jnp.where(data > mid, 1.0, 0.0).astype(jnp.float32),
                      axis=0, keepdims=True)
        ge = cnt >= k_f
        return jnp.where(ge, mid, lo), jnp.where(ge, hi, mid)
    return jax.lax.fori_loop(0, n, body, (lo0, hi0))


def _topk_mean(pre, len_b, l):
    c = pre.shape[1]
    rows = jax.lax.broadcasted_iota(jnp.int32, (l, 1), 0)
    pre_m = jnp.where(rows < len_b, pre, -jnp.inf)
    k_f = ((len_b + jnp.int32(7)) // jnp.int32(8)).astype(jnp.float32)

    lo0 = jnp.full((1, c), -_T_RANGE, jnp.float32)
    hi0 = jnp.full((1, c), _T_RANGE, jnp.float32)
    _, t = _bisect(pre_m, k_f, lo0, hi0, _N_BISECT)

    sig = _sigmoid(pre)
    gt = pre_m > t
    cnt_gt = jnp.sum(gt.astype(jnp.float32), axis=0, keepdims=True)
    sum_gt = jnp.sum(jnp.where(gt, sig, jnp.float32(0.0)), axis=0, keepdims=True)
    t_sig = _sigmoid(t)
    return (sum_gt + (k_f - cnt_gt) * t_sig) / k_f


def _verb_body(lens_ref, e1_ref, w0_ref, w1_ref, w2_ref, bi_ref,
               out_ref, *, l):
    b = pl.program_id(0)
    e1 = e1_ref[0].astype(jnp.bfloat16)
    ym = jnp.dot(e1, w0_ref[...], preferred_element_type=jnp.float32)
    yc = jnp.dot(e1, w1_ref[...], preferred_element_type=jnp.float32)
    yp = jnp.dot(e1, w2_ref[...], preferred_element_type=jnp.float32)
    z = jnp.zeros((1, ym.shape[1]), jnp.float32)
    pre = yc + bi_ref[...] + jnp.concatenate([z, ym[:-1]], axis=0) \
             + jnp.concatenate([yp[1:], z], axis=0)
    out_ref[0] = _topk_mean(pre, lens_ref[b], l)


def _noun_body(lens_ref, e2_ref, w_ref, bi_ref, out_ref, *, l):
    b = pl.program_id(0)
    pre = jnp.dot(e2_ref[0].astype(jnp.bfloat16), w_ref[...],
                  preferred_element_type=jnp.float32) + bi_ref[...]
    out_ref[0] = _topk_mean(pre, lens_ref[b], l)


def kernel(inputs, input_lens, W0, b0, W1, b1, W2, b2, Wa, ba, cvw, cvb,
           bnvg, bnvb, bnvm, bnvv, cnw, cnb, bnng, bnnb, bnnm, bnnv):
    B, L, D_IN = inputs.shape
    D = W0.shape[1]
    VC = cvw.shape[0]
    NC = cnw.shape[0]
    TL = 512 if L % 512 == 0 else L
    NL = L // TL

    lens = input_lens.astype(jnp.int32)

    grid1 = (B, NL)
    full = lambda shp: pl.BlockSpec(shp, lambda b, t: (0,) * len(shp))
    e1, e2, sent = pl.pallas_call(
        functools.partial(_embed_body, tl=TL, nl=NL),
        grid=grid1,
        in_specs=[
            pl.BlockSpec(memory_space=pltpu.SMEM),
            pl.BlockSpec(memory_space=pltpu.SMEM),
            pl.BlockSpec((1, TL, D_IN), lambda b, t: (b, t, 0)),
            full((D_IN, D)), full((1, D)),
            full((D_IN, D)), full((1, D)),
            full((D_IN, D)), full((1, D)),
            full((D, 1)),
        ],
        out_specs=[
            pl.BlockSpec((1, TL, D), lambda b, t: (b, t, 0)),
            pl.BlockSpec((1, TL, D), lambda b, t: (b, t, 0)),
            pl.BlockSpec((1, 1, D), lambda b, t: (b, 0, 0)),
        ],
        out_shape=[
            jax.ShapeDtypeStruct((B, L, D), jnp.float32),
            jax.ShapeDtypeStruct((B, L, D), jnp.float32),
            jax.ShapeDtypeStruct((B, 1, D), jnp.float32),
        ],
        scratch_shapes=[
            pltpu.SMEM((1, 1), jnp.float32),
            pltpu.SMEM((1, 1), jnp.float32),
            pltpu.VMEM((1, D), jnp.float32),
        ],
        compiler_params=pltpu.CompilerParams(
            dimension_semantics=("parallel", "arbitrary")),
    )(lens, ba, inputs, W0.astype(jnp.bfloat16), b0.reshape(1, D),
      W1.astype(jnp.bfloat16), b1.reshape(1, D),
      W2.astype(jnp.bfloat16), b2.reshape(1, D), Wa)

    va = bnvg / jnp.sqrt(bnvv + 1e-5)
    v_scale = va.reshape(1, VC)
    v_bias = (bnvb + (cvb - bnvm) * va).reshape(1, VC)
    na = bnng / jnp.sqrt(bnnv + 1e-5)
    n_scale = na.reshape(1, NC)
    n_bias = (bnnb + (cnb - bnnm) * na).reshape(1, NC)

    wvs = (cvw * va[:, None, None])
    wv0 = wvs[:, :, 0].T.astype(jnp.bfloat16)
    wv1 = wvs[:, :, 1].T.astype(jnp.bfloat16)
    wv2 = wvs[:, :, 2].T.astype(jnp.bfloat16)
    wn = (cnw[:, :, 0] * na[:, None]).T.astype(jnp.bfloat16)

    fullb = lambda shp: pl.BlockSpec(shp, lambda b: (0,) * len(shp))
    ilv = pl.pallas_call(
        functools.partial(_verb_body, l=L),
        grid=(B,),
        in_specs=[
            pl.BlockSpec(memory_space=pltpu.SMEM),
            pl.BlockSpec((1, L, D), lambda b: (b, 0, 0)),
            fullb((D, VC)), fullb((D, VC)), fullb((D, VC)),
            fullb((1, VC)),
        ],
        out_specs=pl.BlockSpec((1, 1, VC), lambda b: (b, 0, 0)),
        out_shape=jax.ShapeDtypeStruct((B, 1, VC), jnp.float32),
        compiler_params=pltpu.CompilerParams(
            dimension_semantics=("parallel",)),
    )(lens, e1, wv0, wv1, wv2, v_bias)

    iln = pl.pallas_call(
        functools.partial(_noun_body, l=L),
        grid=(B,),
        in_specs=[
            pl.BlockSpec(memory_space=pltpu.SMEM),
            pl.BlockSpec((1, L, D), lambda b: (b, 0, 0)),
            fullb((D, NC)), fullb((1, NC)),
        ],
        out_specs=pl.BlockSpec((1, 1, NC), lambda b: (b, 0, 0)),
        out_shape=jax.ShapeDtypeStruct((B, 1, NC), jnp.float32),
        compiler_params=pltpu.CompilerParams(
            dimension_semantics=("parallel",)),
    )(lens, e2, wn, n_bias)

    return (sent.reshape(B, D), e1, e2, ilv.reshape(B, VC), iln.reshape(B, NC))


# speedup vs baseline: 2.6330x; 1.0451x over previous
import functools

import jax
import jax.numpy as jnp
from jax.experimental import pallas as pl
from jax.experimental.pallas import tpu as pltpu

def _embed_body(lens_ref, ba_ref, x_ref, w0_ref, b0_ref, w1_ref, b1_ref,
                w2_ref, b2_ref, wa_ref, e1_ref, e2_ref, sent_ref,
                m_ref, s_ref, acc_ref, *, tl, nl):
    b = pl.program_id(0)
    lt = pl.program_id(1)

    @pl.when(lt == 0)
    def _init():
        m_ref[0, 0] = -jnp.inf
        s_ref[0, 0] = jnp.float32(0.0)
        acc_ref[...] = jnp.zeros_like(acc_ref)

    x = x_ref[0].astype(jnp.bfloat16)
    e0 = jnp.dot(x, w0_ref[...], preferred_element_type=jnp.float32) + b0_ref[...]
    e1 = jnp.dot(x, w1_ref[...], preferred_element_type=jnp.float32) + b1_ref[...]
    e2 = jnp.dot(x, w2_ref[...], preferred_element_type=jnp.float32) + b2_ref[...]
    e1_ref[0] = e1
    e2_ref[0] = e2

    a = jnp.dot(e0, wa_ref[...], preferred_element_type=jnp.float32) + ba_ref[0]
    rows = jax.lax.broadcasted_iota(jnp.int32, (tl, 1), 0) + lt * tl
    a = jnp.where(rows >= lens_ref[b], jnp.float32(-1e18), a)

    m_prev = m_ref[0, 0]
    m_new = jnp.maximum(m_prev, jnp.max(a))
    alpha = jnp.exp(m_prev - m_new)
    p = jnp.exp(a - m_new)
    s_new = s_ref[0, 0] * alpha + jnp.sum(p)
    acc_new = acc_ref[...] * alpha + jnp.sum(p * e0, axis=0, keepdims=True)
    m_ref[0, 0] = m_new
    s_ref[0, 0] = s_new
    acc_ref[...] = acc_new

    @pl.when(lt == nl - 1)
    def _fin():
        sent_ref[0] = acc_new / s_new


_T_RANGE = 30.0
_N_BISECT = 14


def _sigmoid(x):
    return 0.5 + 0.5 * jnp.tanh(0.5 * x)


def _bisect(data, k_f, lo0, hi0, n):
    def body(_, carry):
        lo, hi = carry
        mid = 0.5 * (lo + hi)
        cnt = jnp.sum(jnp.where(data > mid, 1.0, 0.0).astype(jnp.float32),
                      axis=0, keepdims=True)
        ge = cnt >= k_f
        return jnp.where(ge, mid, lo), jnp.where(ge, hi, mid)
    return jax.lax.fori_loop(0, n, body, (lo0, hi0))


def _topk_mean(pre, len_b, l):
    c = pre.shape[1]
    rows = jax.lax.broadcasted_iota(jnp.int32, (l, 1), 0)
    pre_m = jnp.where(rows < len_b, pre, -jnp.inf)
    k_f = ((len_b + jnp.int32(7)) // jnp.int32(8)).astype(jnp.float32)

    lo0 = jnp.full((1, c), -_T_RANGE, jnp.float32)
    hi0 = jnp.full((1, c), _T_RANGE, jnp.float32)
    _, t = _bisect(pre_m, k_f, lo0, hi0, _N_BISECT)

    sig = _sigmoid(pre)
    gt = pre_m > t
    cnt_gt = jnp.sum(gt.astype(jnp.float32), axis=0, keepdims=True)
    sum_gt = jnp.sum(jnp.where(gt, sig, jnp.float32(0.0)), axis=0, keepdims=True)
    t_sig = _sigmoid(t)
    return (sum_gt + (k_f - cnt_gt) * t_sig) / k_f


def _verb_body(lens_ref, e1_ref, w0_ref, w1_ref, w2_ref, bi_ref,
               out_ref, *, l):
    b = pl.program_id(0)
    e1 = e1_ref[0].astype(jnp.bfloat16)
    ym = jnp.dot(e1, w0_ref[...], preferred_element_type=jnp.float32)
    yc = jnp.dot(e1, w1_ref[...], preferred_element_type=jnp.float32)
    yp = jnp.dot(e1, w2_ref[...], preferred_element_type=jnp.float32)
    z = jnp.zeros((1, ym.shape[1]), jnp.float32)
    pre = yc + bi_ref[...] + jnp.concatenate([z, ym[:-1]], axis=0) \
             + jnp.concatenate([yp[1:], z], axis=0)
    out_ref[0] = _topk_mean(pre, lens_ref[b], l)


def _noun_body(lens_ref, e2_ref, w_ref, bi_ref, out_ref, *, l):
    b = pl.program_id(0)
    pre = jnp.dot(e2_ref[0].astype(jnp.bfloat16), w_ref[...],
                  preferred_element_type=jnp.float32) + bi_ref[...]
    out_ref[0] = _topk_mean(pre, lens_ref[b], l)


def kernel(inputs, input_lens, W0, b0, W1, b1, W2, b2, Wa, ba, cvw, cvb,
           bnvg, bnvb, bnvm, bnvv, cnw, cnb, bnng, bnnb, bnnm, bnnv):
    B, L, D_IN = inputs.shape
    D = W0.shape[1]
    VC = cvw.shape[0]
    NC = cnw.shape[0]
    TL = 1024 if L % 1024 == 0 else L
    NL = L // TL

    lens = input_lens.astype(jnp.int32)

    grid1 = (B, NL)
    full = lambda shp: pl.BlockSpec(shp, lambda b, t: (0,) * len(shp))
    e1, e2, sent = pl.pallas_call(
        functools.partial(_embed_body, tl=TL, nl=NL),
        grid=grid1,
        in_specs=[
            pl.BlockSpec(memory_space=pltpu.SMEM),
            pl.BlockSpec(memory_space=pltpu.SMEM),
            pl.BlockSpec((1, TL, D_IN), lambda b, t: (b, t, 0)),
            full((D_IN, D)), full((1, D)),
            full((D_IN, D)), full((1, D)),
            full((D_IN, D)), full((1, D)),
            full((D, 1)),
        ],
        out_specs=[
            pl.BlockSpec((1, TL, D), lambda b, t: (b, t, 0)),
            pl.BlockSpec((1, TL, D), lambda b, t: (b, t, 0)),
            pl.BlockSpec((1, 1, D), lambda b, t: (b, 0, 0)),
        ],
        out_shape=[
            jax.ShapeDtypeStruct((B, L, D), jnp.float32),
            jax.ShapeDtypeStruct((B, L, D), jnp.float32),
            jax.ShapeDtypeStruct((B, 1, D), jnp.float32),
        ],
        scratch_shapes=[
            pltpu.SMEM((1, 1), jnp.float32),
            pltpu.SMEM((1, 1), jnp.float32),
            pltpu.VMEM((1, D), jnp.float32),
        ],
        compiler_params=pltpu.CompilerParams(
            dimension_semantics=("parallel", "arbitrary")),
    )(lens, ba, inputs, W0.astype(jnp.bfloat16), b0.reshape(1, D),
      W1.astype(jnp.bfloat16), b1.reshape(1, D),
      W2.astype(jnp.bfloat16), b2.reshape(1, D), Wa)

    va = bnvg / jnp.sqrt(bnvv + 1e-5)
    v_scale = va.reshape(1, VC)
    v_bias = (bnvb + (cvb - bnvm) * va).reshape(1, VC)
    na = bnng / jnp.sqrt(bnnv + 1e-5)
    n_scale = na.reshape(1, NC)
    n_bias = (bnnb + (cnb - bnnm) * na).reshape(1, NC)

    wvs = (cvw * va[:, None, None])
    wv0 = wvs[:, :, 0].T.astype(jnp.bfloat16)
    wv1 = wvs[:, :, 1].T.astype(jnp.bfloat16)
    wv2 = wvs[:, :, 2].T.astype(jnp.bfloat16)
    wn = (cnw[:, :, 0] * na[:, None]).T.astype(jnp.bfloat16)

    fullb = lambda shp: pl.BlockSpec(shp, lambda b: (0,) * len(shp))
    ilv = pl.pallas_call(
        functools.partial(_verb_body, l=L),
        grid=(B,),
        in_specs=[
            pl.BlockSpec(memory_space=pltpu.SMEM),
            pl.BlockSpec((1, L, D), lambda b: (b, 0, 0)),
            fullb((D, VC)), fullb((D, VC)), fullb((D, VC)),
            fullb((1, VC)),
        ],
        out_specs=pl.BlockSpec((1, 1, VC), lambda b: (b, 0, 0)),
        out_shape=jax.ShapeDtypeStruct((B, 1, VC), jnp.float32),
        compiler_params=pltpu.CompilerParams(
            dimension_semantics=("parallel",)),
    )(lens, e1, wv0, wv1, wv2, v_bias)

    iln = pl.pallas_call(
        functools.partial(_noun_body, l=L),
        grid=(B,),
        in_specs=[
            pl.BlockSpec(memory_space=pltpu.SMEM),
            pl.BlockSpec((1, L, D), lambda b: (b, 0, 0)),
            fullb((D, NC)), fullb((1, NC)),
        ],
        out_specs=pl.BlockSpec((1, 1, NC), lambda b: (b, 0, 0)),
        out_shape=jax.ShapeDtypeStruct((B, 1, NC), jnp.float32),
        compiler_params=pltpu.CompilerParams(
            dimension_semantics=("parallel",)),
    )(lens, e2, wn, n_bias)

    return (sent.reshape(B, D), e1, e2, ilv.reshape(B, VC), iln.reshape(B, NC))


# speedup vs baseline: 2.6587x; 1.0098x over previous
import functools

import jax
import jax.numpy as jnp
from jax.experimental import pallas as pl
from jax.experimental.pallas import tpu as pltpu

def _embed_body(lens_ref, ba_ref, x_ref, w0_ref, b0_ref, w1_ref, b1_ref,
                w2_ref, b2_ref, wa_ref, e1_ref, e2_ref, sent_ref,
                m_ref, s_ref, acc_ref, *, tl, nl):
    b = pl.program_id(0)
    lt = pl.program_id(1)

    @pl.when(lt == 0)
    def _init():
        m_ref[0, 0] = -jnp.inf
        s_ref[0, 0] = jnp.float32(0.0)
        acc_ref[...] = jnp.zeros_like(acc_ref)

    x = x_ref[0].astype(jnp.bfloat16)
    e0 = jnp.dot(x, w0_ref[...], preferred_element_type=jnp.float32) + b0_ref[...]
    e1 = jnp.dot(x, w1_ref[...], preferred_element_type=jnp.float32) + b1_ref[...]
    e2 = jnp.dot(x, w2_ref[...], preferred_element_type=jnp.float32) + b2_ref[...]
    e1_ref[0] = e1
    e2_ref[0] = e2

    a = jnp.dot(e0, wa_ref[...], preferred_element_type=jnp.float32) + ba_ref[0]
    rows = jax.lax.broadcasted_iota(jnp.int32, (tl, 1), 0) + lt * tl
    a = jnp.where(rows >= lens_ref[b], jnp.float32(-1e18), a)

    m_prev = m_ref[0, 0]
    m_new = jnp.maximum(m_prev, jnp.max(a))
    alpha = jnp.exp(m_prev - m_new)
    p = jnp.exp(a - m_new)
    s_new = s_ref[0, 0] * alpha + jnp.sum(p)
    acc_new = acc_ref[...] * alpha + jnp.sum(p * e0, axis=0, keepdims=True)
    m_ref[0, 0] = m_new
    s_ref[0, 0] = s_new
    acc_ref[...] = acc_new

    @pl.when(lt == nl - 1)
    def _fin():
        sent_ref[0] = acc_new / s_new


_T_RANGE = 30.0
_N_BISECT = 14


def _sigmoid(x):
    return 0.5 + 0.5 * jnp.tanh(0.5 * x)


def _bisect(data, k_f, lo0, hi0, n):
    def body(_, carry):
        lo, hi = carry
        mid = 0.5 * (lo + hi)
        cnt = jnp.sum(jnp.where(data > mid, 1.0, 0.0).astype(jnp.float32),
                      axis=0, keepdims=True)
        ge = cnt >= k_f
        return jnp.where(ge, mid, lo), jnp.where(ge, hi, mid)
    return jax.lax.fori_loop(0, n, body, (lo0, hi0))


def _topk_mean(pre, len_b, l):
    c = pre.shape[1]
    rows = jax.lax.broadcasted_iota(jnp.int32, (l, 1), 0)
    pre_m = jnp.where(rows < len_b, pre, -jnp.inf)
    k_f = ((len_b + jnp.int32(7)) // jnp.int32(8)).astype(jnp.float32)

    lo0 = jnp.full((1, c), -_T_RANGE, jnp.float32)
    hi0 = jnp.full((1, c), _T_RANGE, jnp.float32)
    _, t = _bisect(pre_m, k_f, lo0, hi0, _N_BISECT)

    sig = _sigmoid(pre)
    gt = pre_m > t
    cnt_gt = jnp.sum(gt.astype(jnp.float32), axis=0, keepdims=True)
    sum_gt = jnp.sum(jnp.where(gt, sig, jnp.float32(0.0)), axis=0, keepdims=True)
    t_sig = _sigmoid(t)
    return (sum_gt + (k_f - cnt_gt) * t_sig) / k_f


def _cls_body(lens_ref, e1_ref, e2_ref, w0_ref, w1_ref, w2_ref, vb_ref,
              wn_ref, nb_ref, ilv_ref, iln_ref, *, l):
    b = pl.program_id(0)
    len_b = lens_ref[b]
    e1 = e1_ref[0].astype(jnp.bfloat16)
    ym = jnp.dot(e1, w0_ref[...], preferred_element_type=jnp.float32)
    yc = jnp.dot(e1, w1_ref[...], preferred_element_type=jnp.float32)
    yp = jnp.dot(e1, w2_ref[...], preferred_element_type=jnp.float32)
    z = jnp.zeros((1, ym.shape[1]), jnp.float32)
    pre_v = yc + vb_ref[...] + jnp.concatenate([z, ym[:-1]], axis=0) \
                + jnp.concatenate([yp[1:], z], axis=0)
    ilv_ref[0] = _topk_mean(pre_v, len_b, l)
    pre_n = jnp.dot(e2_ref[0].astype(jnp.bfloat16), wn_ref[...],
                    preferred_element_type=jnp.float32) + nb_ref[...]
    iln_ref[0] = _topk_mean(pre_n, len_b, l)


def kernel(inputs, input_lens, W0, b0, W1, b1, W2, b2, Wa, ba, cvw, cvb,
           bnvg, bnvb, bnvm, bnvv, cnw, cnb, bnng, bnnb, bnnm, bnnv):
    B, L, D_IN = inputs.shape
    D = W0.shape[1]
    VC = cvw.shape[0]
    NC = cnw.shape[0]
    TL = 1024 if L % 1024 == 0 else L
    NL = L // TL

    lens = input_lens.astype(jnp.int32)

    grid1 = (B, NL)
    full = lambda shp: pl.BlockSpec(shp, lambda b, t: (0,) * len(shp))
    e1, e2, sent = pl.pallas_call(
        functools.partial(_embed_body, tl=TL, nl=NL),
        grid=grid1,
        in_specs=[
            pl.BlockSpec(memory_space=pltpu.SMEM),
            pl.BlockSpec(memory_space=pltpu.SMEM),
            pl.BlockSpec((1, TL, D_IN), lambda b, t: (b, t, 0)),
            full((D_IN, D)), full((1, D)),
            full((D_IN, D)), full((1, D)),
            full((D_IN, D)), full((1, D)),
            full((D, 1)),
        ],
        out_specs=[
            pl.BlockSpec((1, TL, D), lambda b, t: (b, t, 0)),
            pl.BlockSpec((1, TL, D), lambda b, t: (b, t, 0)),
            pl.BlockSpec((1, 1, D), lambda b, t: (b, 0, 0)),
        ],
        out_shape=[
            jax.ShapeDtypeStruct((B, L, D), jnp.float32),
            jax.ShapeDtypeStruct((B, L, D), jnp.float32),
            jax.ShapeDtypeStruct((B, 1, D), jnp.float32),
        ],
        scratch_shapes=[
            pltpu.SMEM((1, 1), jnp.float32),
            pltpu.SMEM((1, 1), jnp.float32),
            pltpu.VMEM((1, D), jnp.float32),
        ],
        compiler_params=pltpu.CompilerParams(
            dimension_semantics=("parallel", "arbitrary")),
    )(lens, ba, inputs, W0.astype(jnp.bfloat16), b0.reshape(1, D),
      W1.astype(jnp.bfloat16), b1.reshape(1, D),
      W2.astype(jnp.bfloat16), b2.reshape(1, D), Wa)

    va = bnvg / jnp.sqrt(bnvv + 1e-5)
    v_scale = va.reshape(1, VC)
    v_bias = (bnvb + (cvb - bnvm) * va).reshape(1, VC)
    na = bnng / jnp.sqrt(bnnv + 1e-5)
    n_scale = na.reshape(1, NC)
    n_bias = (bnnb + (cnb - bnnm) * na).reshape(1, NC)

    wvs = (cvw * va[:, None, None])
    wv0 = wvs[:, :, 0].T.astype(jnp.bfloat16)
    wv1 = wvs[:, :, 1].T.astype(jnp.bfloat16)
    wv2 = wvs[:, :, 2].T.astype(jnp.bfloat16)
    wn = (cnw[:, :, 0] * na[:, None]).T.astype(jnp.bfloat16)

    fullb = lambda shp: pl.BlockSpec(shp, lambda b: (0,) * len(shp))
    ilv, iln = pl.pallas_call(
        functools.partial(_cls_body, l=L),
        grid=(B,),
        in_specs=[
            pl.BlockSpec(memory_space=pltpu.SMEM),
            pl.BlockSpec((1, L, D), lambda b: (b, 0, 0)),
            pl.BlockSpec((1, L, D), lambda b: (b, 0, 0)),
            fullb((D, VC)), fullb((D, VC)), fullb((D, VC)),
            fullb((1, VC)),
            fullb((D, NC)), fullb((1, NC)),
        ],
        out_specs=[
            pl.BlockSpec((1, 1, VC), lambda b: (b, 0, 0)),
            pl.BlockSpec((1, 1, NC), lambda b: (b, 0, 0)),
        ],
        out_shape=[
            jax.ShapeDtypeStruct((B, 1, VC), jnp.float32),
            jax.ShapeDtypeStruct((B, 1, NC), jnp.float32),
        ],
        compiler_params=pltpu.CompilerParams(
            dimension_semantics=("parallel",)),
    )(lens, e1, e2, wv0, wv1, wv2, v_bias, wn, n_bias)

    return (sent.reshape(B, D), e1, e2, ilv.reshape(B, VC), iln.reshape(B, NC))


# speedup vs baseline: 2.8109x; 1.0573x over previous
import functools

import jax
import jax.numpy as jnp
from jax.experimental import pallas as pl
from jax.experimental.pallas import tpu as pltpu

def _embed_body(lens_ref, ba_ref, x_ref, w0_ref, b0_ref, w1_ref, b1_ref,
                w2_ref, b2_ref, wa_ref, e1_ref, e2_ref, sent_ref,
                m_ref, s_ref, acc_ref, *, tl, nl):
    b = pl.program_id(0)
    lt = pl.program_id(1)

    @pl.when(lt == 0)
    def _init():
        m_ref[0, 0] = -jnp.inf
        s_ref[0, 0] = jnp.float32(0.0)
        acc_ref[...] = jnp.zeros_like(acc_ref)

    x = x_ref[0].astype(jnp.bfloat16)
    e0 = jnp.dot(x, w0_ref[...], preferred_element_type=jnp.float32) + b0_ref[...]
    e1 = jnp.dot(x, w1_ref[...], preferred_element_type=jnp.float32) + b1_ref[...]
    e2 = jnp.dot(x, w2_ref[...], preferred_element_type=jnp.float32) + b2_ref[...]
    e1_ref[0] = e1
    e2_ref[0] = e2

    a = jnp.dot(e0, wa_ref[...], preferred_element_type=jnp.float32) + ba_ref[0]
    rows = jax.lax.broadcasted_iota(jnp.int32, (tl, 1), 0) + lt * tl
    a = jnp.where(rows >= lens_ref[b], jnp.float32(-1e18), a)

    m_prev = m_ref[0, 0]
    m_new = jnp.maximum(m_prev, jnp.max(a))
    alpha = jnp.exp(m_prev - m_new)
    p = jnp.exp(a - m_new)
    s_new = s_ref[0, 0] * alpha + jnp.sum(p)
    acc_new = acc_ref[...] * alpha + jnp.sum(p * e0, axis=0, keepdims=True)
    m_ref[0, 0] = m_new
    s_ref[0, 0] = s_new
    acc_ref[...] = acc_new

    @pl.when(lt == nl - 1)
    def _fin():
        sent_ref[0] = acc_new / s_new


_T_RANGE = 30.0
_N_BISECT = 13


def _sigmoid(x):
    return 0.5 + 0.5 * jnp.tanh(0.5 * x)


def _bisect(data, k_f, lo0, hi0, n):
    def body(_, carry):
        lo, hi = carry
        mid = 0.5 * (lo + hi)
        cnt = jnp.sum(jnp.where(data > mid, 1.0, 0.0).astype(jnp.float32),
                      axis=0, keepdims=True)
        ge = cnt >= k_f
        return jnp.where(ge, mid, lo), jnp.where(ge, hi, mid)
    return jax.lax.fori_loop(0, n, body, (lo0, hi0))


def _topk_mean(pre, len_b, l):
    c = pre.shape[1]
    rows = jax.lax.broadcasted_iota(jnp.int32, (l, 1), 0)
    pre_m = jnp.where(rows < len_b, pre, -jnp.inf)
    k_f = ((len_b + jnp.int32(7)) // jnp.int32(8)).astype(jnp.float32)

    lo0 = jnp.full((1, c), -_T_RANGE, jnp.float32)
    hi0 = jnp.full((1, c), _T_RANGE, jnp.float32)
    _, t = _bisect(pre_m, k_f, lo0, hi0, _N_BISECT)

    t_sig = _sigmoid(t)
    fill = jnp.maximum(_sigmoid(pre_m) - t_sig, jnp.float32(0.0))
    return jnp.sum(fill, axis=0, keepdims=True) / k_f + t_sig


def _cls_body(lens_ref, e1_ref, e2_ref, w0_ref, w1_ref, w2_ref, vb_ref,
              wn_ref, nb_ref, ilv_ref, iln_ref, *, l):
    b = pl.program_id(0)
    len_b = lens_ref[b]
    e1 = e1_ref[0].astype(jnp.bfloat16)
    ym = jnp.dot(e1, w0_ref[...], preferred_element_type=jnp.float32)
    yc = jnp.dot(e1, w1_ref[...], preferred_element_type=jnp.float32)
    yp = jnp.dot(e1, w2_ref[...], preferred_element_type=jnp.float32)
    z = jnp.zeros((1, ym.shape[1]), jnp.float32)
    pre_v = yc + vb_ref[...] + jnp.concatenate([z, ym[:-1]], axis=0) \
                + jnp.concatenate([yp[1:], z], axis=0)
    ilv_ref[0] = _topk_mean(pre_v, len_b, l)
    pre_n = jnp.dot(e2_ref[0].astype(jnp.bfloat16), wn_ref[...],
                    preferred_element_type=jnp.float32) + nb_ref[...]
    iln_ref[0] = _topk_mean(pre_n, len_b, l)


def kernel(inputs, input_lens, W0, b0, W1, b1, W2, b2, Wa, ba, cvw, cvb,
           bnvg, bnvb, bnvm, bnvv, cnw, cnb, bnng, bnnb, bnnm, bnnv):
    B, L, D_IN = inputs.shape
    D = W0.shape[1]
    VC = cvw.shape[0]
    NC = cnw.shape[0]
    TL = 1024 if L % 1024 == 0 else L
    NL = L // TL

    lens = input_lens.astype(jnp.int32)

    grid1 = (B, NL)
    full = lambda shp: pl.BlockSpec(shp, lambda b, t: (0,) * len(shp))
    e1, e2, sent = pl.pallas_call(
        functools.partial(_embed_body, tl=TL, nl=NL),
        grid=grid1,
        in_specs=[
            pl.BlockSpec(memory_space=pltpu.SMEM),
            pl.BlockSpec(memory_space=pltpu.SMEM),
            pl.BlockSpec((1, TL, D_IN), lambda b, t: (b, t, 0)),
            full((D_IN, D)), full((1, D)),
            full((D_IN, D)), full((1, D)),
            full((D_IN, D)), full((1, D)),
            full((D, 1)),
        ],
        out_specs=[
            pl.BlockSpec((1, TL, D), lambda b, t: (b, t, 0)),
            pl.BlockSpec((1, TL, D), lambda b, t: (b, t, 0)),
            pl.BlockSpec((1, 1, D), lambda b, t: (b, 0, 0)),
        ],
        out_shape=[
            jax.ShapeDtypeStruct((B, L, D), jnp.float32),
            jax.ShapeDtypeStruct((B, L, D), jnp.float32),
            jax.ShapeDtypeStruct((B, 1, D), jnp.float32),
        ],
        scratch_shapes=[
            pltpu.SMEM((1, 1), jnp.float32),
            pltpu.SMEM((1, 1), jnp.float32),
            pltpu.VMEM((1, D), jnp.float32),
        ],
        compiler_params=pltpu.CompilerParams(
            dimension_semantics=("parallel", "arbitrary")),
    )(lens, ba, inputs, W0.astype(jnp.bfloat16), b0.reshape(1, D),
      W1.astype(jnp.bfloat16), b1.reshape(1, D),
      W2.astype(jnp.bfloat16), b2.reshape(1, D), Wa)

    va = bnvg / jnp.sqrt(bnvv + 1e-5)
    v_scale = va.reshape(1, VC)
    v_bias = (bnvb + (cvb - bnvm) * va).reshape(1, VC)
    na = bnng / jnp.sqrt(bnnv + 1e-5)
    n_scale = na.reshape(1, NC)
    n_bias = (bnnb + (cnb - bnnm) * na).reshape(1, NC)

    wvs = (cvw * va[:, None, None])
    wv0 = wvs[:, :, 0].T.astype(jnp.bfloat16)
    wv1 = wvs[:, :, 1].T.astype(jnp.bfloat16)
    wv2 = wvs[:, :, 2].T.astype(jnp.bfloat16)
    wn = (cnw[:, :, 0] * na[:, None]).T.astype(jnp.bfloat16)

    fullb = lambda shp: pl.BlockSpec(shp, lambda b: (0,) * len(shp))
    ilv, iln = pl.pallas_call(
        functools.partial(_cls_body, l=L),
        grid=(B,),
        in_specs=[
            pl.BlockSpec(memory_space=pltpu.SMEM),
            pl.BlockSpec((1, L, D), lambda b: (b, 0, 0)),
            pl.BlockSpec((1, L, D), lambda b: (b, 0, 0)),
            fullb((D, VC)), fullb((D, VC)), fullb((D, VC)),
            fullb((1, VC)),
            fullb((D, NC)), fullb((1, NC)),
        ],
        out_specs=[
            pl.BlockSpec((1, 1, VC), lambda b: (b, 0, 0)),
            pl.BlockSpec((1, 1, NC), lambda b: (b, 0, 0)),
        ],
        out_shape=[
            jax.ShapeDtypeStruct((B, 1, VC), jnp.float32),
            jax.ShapeDtypeStruct((B, 1, NC), jnp.float32),
        ],
        compiler_params=pltpu.CompilerParams(
            dimension_semantics=("parallel",)),
    )(lens, e1, e2, wv0, wv1, wv2, v_bias, wn, n_bias)

    return (sent.reshape(B, D), e1, e2, ilv.reshape(B, VC), iln.reshape(B, NC))
